# baseline (speedup 1.0000x reference)
import functools
import math

import jax
import jax.numpy as jnp
from jax import lax
from jax.experimental import pallas as pl
from jax.experimental.pallas import tpu as pltpu
from jax.experimental.pallas import tpu_sc as plsc

N = 10000
C = 256
K = 16
H = 8
DH = C // H
FF = 512
OUT = 256


_NB_A = 1000


def _proj_body(vf_ref, cp_ref, wqp_ref, bqp_ref, wkp_ref, bkp_ref,
               wq_ref, bq_ref, wk_ref, bk_ref, wv_ref, bv_ref,
               q_ref, kv_ref):
    vf = vf_ref[...]
    cp = cp_ref[...]
    posq = jnp.maximum(
        jnp.dot(cp, wqp_ref[...], preferred_element_type=jnp.float32)
        + bqp_ref[...], 0.0)
    posk = jnp.maximum(
        jnp.dot(cp, wkp_ref[...], preferred_element_type=jnp.float32)
        + bkp_ref[...], 0.0)
    qf = vf + posq
    kf = vf + posk
    q_ref[...] = (jnp.dot(qf, wq_ref[...], preferred_element_type=jnp.float32)
                  + bq_ref[...])
    kmat = (jnp.dot(kf, wk_ref[...], preferred_element_type=jnp.float32)
            + bk_ref[...])
    vmat = (jnp.dot(kf, wv_ref[...], preferred_element_type=jnp.float32)
            + bv_ref[...])

    def bf16_bits(x):
        u = jax.lax.bitcast_convert_type(x, jnp.uint32)
        lsb = (u >> 16) & jnp.uint32(1)
        return (u + jnp.uint32(0x7FFF) + lsb) >> 16

    packed = (bf16_bits(vmat) << 16) | bf16_bits(kmat)
    kv_ref[...] = jax.lax.bitcast_convert_type(packed, jnp.int32)


def _stage_a(vf, cp, wqp, bqp, wkp, bkp, wq, bq, wk, bk, wv, bv):
    nsteps = N // _NB_A
    row = pl.BlockSpec((_NB_A, None), lambda i: (i, 0))

    def full(shape):
        return pl.BlockSpec(shape, lambda i: tuple(0 for _ in shape))

    return pl.pallas_call(
        _proj_body,
        grid=(nsteps,),
        in_specs=[
            pl.BlockSpec((_NB_A, C), lambda i: (i, 0)),
            pl.BlockSpec((_NB_A, 8), lambda i: (i, 0)),
            full((8, C)), full((1, C)),
            full((8, C)), full((1, C)),
            full((C, C)), full((1, C)),
            full((C, C)), full((1, C)),
            full((C, C)), full((1, C)),
        ],
        out_specs=[
            pl.BlockSpec((_NB_A, C), lambda i: (i, 0)),
            pl.BlockSpec((_NB_A, C), lambda i: (i, 0)),
        ],
        out_shape=[
            jax.ShapeDtypeStruct((N, C), jnp.float32),
            jax.ShapeDtypeStruct((N, C), jnp.int32),
        ],
    )(vf, cp, wqp, bqp, wkp, bkp, wq, bq, wk, bk, wv, bv)


_NW = 32
_CH = 200


def _sc_gather(table, idx):
    nrows = idx.shape[0]
    rw = nrows // _NW
    ch = max(c for c in range(8, 248, 8) if rw % c == 0)
    nch = rw // ch
    assert rw % 8 == 0
    mesh = plsc.VectorSubcoreMesh(core_axis_name="c", subcore_axis_name="s")

    @functools.partial(
        pl.kernel,
        out_type=jax.ShapeDtypeStruct((nrows, C), jnp.int32),
        mesh=mesh,
        scratch_types=[
            pltpu.VMEM((rw,), jnp.int32),
            pltpu.VMEM((ch, C), jnp.int32),
            pltpu.VMEM((ch, C), jnp.int32),
            pltpu.SemaphoreType.DMA,
            pltpu.SemaphoreType.DMA,
            pltpu.SemaphoreType.DMA,
            pltpu.SemaphoreType.DMA,
        ],
    )
    def gather_kernel(table_hbm, idx_hbm, out_hbm, idx_all, buf0, buf1,
                      gsem0, gsem1, ssem0, ssem1):
        wid = lax.axis_index("s") * 2 + lax.axis_index("c")
        base = wid * rw
        pltpu.sync_copy(idx_hbm.at[pl.ds(base, rw)], idx_all)
        bufs = (buf0, buf1)
        gsems = (gsem0, gsem1)
        ssems = (ssem0, ssem1)
        stores = [None, None]
        for i in range(nch):
            b = i % 2
            if stores[b] is not None:
                stores[b].wait()
            g = pltpu.async_copy(
                table_hbm.at[idx_all.at[pl.ds(i * ch, ch)]],
                bufs[b], gsems[b])
            g.wait()
            stores[b] = pltpu.async_copy(
                bufs[b], out_hbm.at[pl.ds(base + i * ch, ch)], ssems[b])
        stores[0].wait()
        stores[1].wait()

    return gather_kernel(table, idx)


_NB_C = 1000


def _attn_body(q_ref, kvg_ref, mask_ref, vf_ref, wo_ref, bo_ref,
               x1_ref, st_ref):
    nb = _NB_C
    q = q_ref[...]
    p = kvg_ref[...]
    kg = jax.lax.bitcast_convert_type(p << 16, jnp.float32)
    vg = jax.lax.bitcast_convert_type(p & jnp.int32(-65536), jnp.float32)
    q3 = jnp.broadcast_to(q[:, None, :], (nb, K, C)).reshape(nb * K, C)
    prod = q3 * kg
    ci = lax.broadcasted_iota(jnp.int32, (C, H), 0)
    hi = lax.broadcasted_iota(jnp.int32, (C, H), 1)
    bmat = ((ci // DH == hi).astype(jnp.float32)
            * jnp.float32(1.0 / math.sqrt(DH)))
    s0 = jnp.dot(prod, bmat, preferred_element_type=jnp.float32)
    s3 = s0.reshape(nb, K, H)
    mask = mask_ref[...]
    s3 = jnp.where(mask[:, :, None] != 0.0, jnp.float32(-86.0), s3)
    e = jnp.exp(s3)
    attn = e / jnp.sum(e, axis=1, keepdims=True)
    hj = lax.broadcasted_iota(jnp.int32, (H, C), 0)
    cj = lax.broadcasted_iota(jnp.int32, (H, C), 1)
    bmat_t = (cj // DH == hj).astype(jnp.float32)
    ae = jnp.dot(attn.reshape(nb * K, H), bmat_t,
                 preferred_element_type=jnp.float32)
    attend = jnp.sum((ae * vg).reshape(nb, K, C), axis=1)
    x1 = (vf_ref[...]
          + jnp.dot(attend, wo_ref[...], preferred_element_type=jnp.float32)
          + bo_ref[...])
    x1_ref[...] = x1

    @pl.when(pl.program_id(0) == 0)
    def _():
        st_ref[...] = jnp.zeros_like(st_ref)

    st_ref[0:1, :] += jnp.sum(x1, axis=0, keepdims=True)
    st_ref[1:2, :] += jnp.sum(x1 * x1, axis=0, keepdims=True)


def _stage_c(q, kvg, maskf, vf, wo, bo, v_start, v_count):
    nsteps = v_count // _NB_C
    b0 = v_start // _NB_C

    def full(shape):
        return pl.BlockSpec(shape, lambda i: tuple(0 for _ in shape))

    return pl.pallas_call(
        _attn_body,
        grid=(nsteps,),
        in_specs=[
            pl.BlockSpec((_NB_C, C), lambda i: (b0 + i, 0)),
            pl.BlockSpec((_NB_C * K, C), lambda i: (i, 0)),
            pl.BlockSpec((_NB_C, K), lambda i: (b0 + i, 0)),
            pl.BlockSpec((_NB_C, C), lambda i: (b0 + i, 0)),
            full((C, C)), full((1, C)),
        ],
        out_specs=[
            pl.BlockSpec((_NB_C, C), lambda i: (i, 0)),
            full((8, C)),
        ],
        out_shape=[
            jax.ShapeDtypeStruct((v_count, C), jnp.float32),
            jax.ShapeDtypeStruct((8, C), jnp.float32),
        ],
    )(q, kvg, maskf, vf, wo, bo)


_NB_D = 1000


def _ffn_body(x1_ref, s1_ref, g1_ref, be1_ref, w1_ref, b1_ref, w2_ref, b2_ref,
              x2_ref, st_ref):
    invn = jnp.float32(1.0 / N)
    mean = s1_ref[0:1, :] * invn
    var = s1_ref[1:2, :] * invn - mean * mean
    inv = lax.rsqrt(var + 1e-5)
    x1n = (x1_ref[...] - mean) * inv * g1_ref[...] + be1_ref[...]
    hmid = jnp.maximum(
        jnp.dot(x1n, w1_ref[...], preferred_element_type=jnp.float32)
        + b1_ref[...], 0.0)
    ff = (jnp.dot(hmid, w2_ref[...], preferred_element_type=jnp.float32)
          + b2_ref[...])
    x2 = x1n + ff
    x2_ref[...] = x2

    @pl.when(pl.program_id(0) == 0)
    def _():
        st_ref[...] = jnp.zeros_like(st_ref)

    st_ref[0:1, :] += jnp.sum(x2, axis=0, keepdims=True)
    st_ref[1:2, :] += jnp.sum(x2 * x2, axis=0, keepdims=True)


def _stage_d(x1, s1, g1, be1, w1, b1, w2, b2):
    nsteps = N // _NB_D

    def full(shape):
        return pl.BlockSpec(shape, lambda i: tuple(0 for _ in shape))

    return pl.pallas_call(
        _ffn_body,
        grid=(nsteps,),
        in_specs=[
            pl.BlockSpec((_NB_D, C), lambda i: (i, 0)),
            full((8, C)), full((1, C)), full((1, C)),
            full((C, FF)), full((1, FF)),
            full((FF, C)), full((1, C)),
        ],
        out_specs=[
            pl.BlockSpec((_NB_D, C), lambda i: (i, 0)),
            full((8, C)),
        ],
        out_shape=[
            jax.ShapeDtypeStruct((N, C), jnp.float32),
            jax.ShapeDtypeStruct((8, C), jnp.float32),
        ],
    )(x1, s1, g1, be1, w1, b1, w2, b2)


def _outproj_body(x2_ref, s2_ref, g2_ref, be2_ref, wout_ref, bout_ref,
                  y_ref, st_ref):
    invn = jnp.float32(1.0 / N)
    mean = s2_ref[0:1, :] * invn
    var = s2_ref[1:2, :] * invn - mean * mean
    inv = lax.rsqrt(var + 1e-5)
    x2n = (x2_ref[...] - mean) * inv * g2_ref[...] + be2_ref[...]
    y = (jnp.dot(x2n, wout_ref[...], preferred_element_type=jnp.float32)
         + bout_ref[...])
    y_ref[...] = y

    @pl.when(pl.program_id(0) == 0)
    def _():
        st_ref[...] = jnp.zeros_like(st_ref)

    st_ref[0:1, :] += jnp.sum(y, axis=0, keepdims=True)
    st_ref[1:2, :] += jnp.sum(y * y, axis=0, keepdims=True)


def _stage_e(x2, s2, g2, be2, wout, bout):
    nsteps = N // _NB_D

    def full(shape):
        return pl.BlockSpec(shape, lambda i: tuple(0 for _ in shape))

    return pl.pallas_call(
        _outproj_body,
        grid=(nsteps,),
        in_specs=[
            pl.BlockSpec((_NB_D, C), lambda i: (i, 0)),
            full((8, C)), full((1, C)), full((1, C)),
            full((C, OUT)), full((1, OUT)),
        ],
        out_specs=[
            pl.BlockSpec((_NB_D, OUT), lambda i: (i, 0)),
            full((8, OUT)),
        ],
        out_shape=[
            jax.ShapeDtypeStruct((N, OUT), jnp.float32),
            jax.ShapeDtypeStruct((8, OUT), jnp.float32),
        ],
    )(x2, s2, g2, be2, wout, bout)


def _final_body(y_ref, s3_ref, gout_ref, beout_ref, out_ref):
    invn = jnp.float32(1.0 / N)
    mean = s3_ref[0:1, :] * invn
    var = s3_ref[1:2, :] * invn - mean * mean
    inv = lax.rsqrt(var + 1e-5)
    yn = (y_ref[...] - mean) * inv * gout_ref[...] + beout_ref[...]
    out_ref[...] = jnp.maximum(yn, 0.0)


def _stage_f(y, s3, gout, beout):
    nsteps = N // _NB_D

    def full(shape):
        return pl.BlockSpec(shape, lambda i: tuple(0 for _ in shape))

    return pl.pallas_call(
        _final_body,
        grid=(nsteps,),
        in_specs=[
            pl.BlockSpec((_NB_D, OUT), lambda i: (i, 0)),
            full((8, OUT)), full((1, OUT)), full((1, OUT)),
        ],
        out_specs=pl.BlockSpec((_NB_D, OUT), lambda i: (i, 0)),
        out_shape=jax.ShapeDtypeStruct((N, OUT), jnp.float32),
    )(y, s3, gout, beout)


def kernel(voxel_features, voxel_coords, key_indices, key_mask, Wq, bq, Wk, bk,
           Wv, bv, Wo, bo, W1, b1, W2, b2, Wqp, bqp, Wkp, bkp, Wout, bout,
           g1, be1, g2, be2, gout, beout):
    cp = jnp.pad(voxel_coords, ((0, 0), (0, 5)))
    wqp = jnp.pad(Wqp, ((0, 5), (0, 0)))
    wkp = jnp.pad(Wkp, ((0, 5), (0, 0)))
    r = lambda b: b.reshape(1, -1)

    q, kv = _stage_a(voxel_features, cp, wqp, r(bqp), wkp, r(bkp),
                     Wq, r(bq), Wk, r(bk), Wv, r(bv))
    idx = key_indices.reshape(-1).astype(jnp.int32)
    splits = [(0, N)]
    kvgs = [_sc_gather(kv, idx[s * K:(s + c) * K]) for s, c in splits]
    maskf = key_mask.astype(jnp.float32)
    parts = [_stage_c(q, kvg, maskf, voxel_features, Wo, r(bo), s, c)
             for kvg, (s, c) in zip(kvgs, splits)]
    x1 = (parts[0][0] if len(parts) == 1
          else jnp.concatenate([p[0] for p in parts], axis=0))
    s1 = parts[0][1]
    for p in parts[1:]:
        s1 = s1 + p[1]
    x2, s2 = _stage_d(x1, s1, g1.reshape(1, -1), be1.reshape(1, -1),
                      W1, r(b1), W2, r(b2))
    y, s3 = _stage_e(x2, s2, g2.reshape(1, -1), be2.reshape(1, -1),
                     Wout, r(bout))
    return _stage_f(y, s3, gout.reshape(1, -1), beout.reshape(1, -1))


# speedup vs baseline: 1.0113x; 1.0113x over previous
import functools
import math

import jax
import jax.numpy as jnp
from jax import lax
from jax.experimental import pallas as pl
from jax.experimental.pallas import tpu as pltpu
from jax.experimental.pallas import tpu_sc as plsc

N = 10000
C = 256
K = 16
H = 8
DH = C // H
FF = 512
OUT = 256


_NB_A = 1000


def _proj_body(vf_ref, cp_ref, wqp_ref, bqp_ref, wkp_ref, bkp_ref,
               wq_ref, bq_ref, wk_ref, bk_ref, wv_ref, bv_ref,
               q_ref, kv_ref):
    vf = vf_ref[...]
    cp = cp_ref[...]
    posq = jnp.maximum(
        jnp.dot(cp, wqp_ref[...], preferred_element_type=jnp.float32)
        + bqp_ref[...], 0.0)
    posk = jnp.maximum(
        jnp.dot(cp, wkp_ref[...], preferred_element_type=jnp.float32)
        + bkp_ref[...], 0.0)
    qf = vf + posq
    kf = vf + posk
    q_ref[...] = (jnp.dot(qf, wq_ref[...], preferred_element_type=jnp.float32)
                  + bq_ref[...])
    kmat = (jnp.dot(kf, wk_ref[...], preferred_element_type=jnp.float32)
            + bk_ref[...])
    vmat = (jnp.dot(kf, wv_ref[...], preferred_element_type=jnp.float32)
            + bv_ref[...])

    def bf16_bits(x):
        u = jax.lax.bitcast_convert_type(x, jnp.uint32)
        lsb = (u >> 16) & jnp.uint32(1)
        return (u + jnp.uint32(0x7FFF) + lsb) >> 16

    packed = (bf16_bits(vmat) << 16) | bf16_bits(kmat)
    kv_ref[...] = jax.lax.bitcast_convert_type(packed, jnp.int32)


def _stage_a(vf, cp, wqp, bqp, wkp, bkp, wq, bq, wk, bk, wv, bv):
    nsteps = N // _NB_A

    def full(shape):
        return pl.BlockSpec(shape, lambda i: tuple(0 for _ in shape))

    return pl.pallas_call(
        _proj_body,
        grid=(nsteps,),
        in_specs=[
            pl.BlockSpec((_NB_A, C), lambda i: (i, 0)),
            pl.BlockSpec((_NB_A, 8), lambda i: (i, 0)),
            full((8, C)), full((1, C)),
            full((8, C)), full((1, C)),
            full((C, C)), full((1, C)),
            full((C, C)), full((1, C)),
            full((C, C)), full((1, C)),
        ],
        out_specs=[
            pl.BlockSpec((_NB_A, C), lambda i: (i, 0)),
            pl.BlockSpec((_NB_A, C), lambda i: (i, 0)),
        ],
        out_shape=[
            jax.ShapeDtypeStruct((N, C), jnp.float32),
            jax.ShapeDtypeStruct((N, C), jnp.int32),
        ],
    )(vf, cp, wqp, bqp, wkp, bkp, wq, bq, wk, bk, wv, bv)


_NW = 32


def _sc_gather(table, idx):
    nrows = idx.shape[0]
    rw = nrows // _NW
    ch = max(c for c in range(8, 248, 8) if rw % c == 0)
    nch = rw // ch
    assert rw % 8 == 0
    mesh = plsc.VectorSubcoreMesh(core_axis_name="c", subcore_axis_name="s")

    @functools.partial(
        pl.kernel,
        out_type=jax.ShapeDtypeStruct((nrows, C), jnp.int32),
        mesh=mesh,
        scratch_types=[
            pltpu.VMEM((rw,), jnp.int32),
            pltpu.VMEM((ch, C), jnp.int32),
            pltpu.VMEM((ch, C), jnp.int32),
            pltpu.SemaphoreType.DMA,
            pltpu.SemaphoreType.DMA,
            pltpu.SemaphoreType.DMA,
            pltpu.SemaphoreType.DMA,
        ],
    )
    def gather_kernel(table_hbm, idx_hbm, out_hbm, idx_all, buf0, buf1,
                      gsem0, gsem1, ssem0, ssem1):
        wid = lax.axis_index("s") * 2 + lax.axis_index("c")
        base = wid * rw
        pltpu.sync_copy(idx_hbm.at[pl.ds(base, rw)], idx_all)
        bufs = (buf0, buf1)
        gsems = (gsem0, gsem1)
        ssems = (ssem0, ssem1)
        stores = [None, None]
        for i in range(nch):
            b = i % 2
            if stores[b] is not None:
                stores[b].wait()
            g = pltpu.async_copy(
                table_hbm.at[idx_all.at[pl.ds(i * ch, ch)]],
                bufs[b], gsems[b])
            g.wait()
            stores[b] = pltpu.async_copy(
                bufs[b], out_hbm.at[pl.ds(base + i * ch, ch)], ssems[b])
        stores[0].wait()
        stores[1].wait()

    return gather_kernel(table, idx)


_NB_C = 400


def _attn_body(q_ref, kvg_ref, mask_ref, vf_ref, wo_ref, bo_ref,
               x1_ref, st_ref):
    nb = _NB_C
    q = q_ref[...]
    p = kvg_ref[...]
    kg = jax.lax.bitcast_convert_type(p << 16, jnp.float32)
    vg = jax.lax.bitcast_convert_type(p & jnp.int32(-65536), jnp.float32)
    q3 = jnp.broadcast_to(q[:, None, :], (nb, K, C)).reshape(nb * K, C)
    prod = q3 * kg
    ci = lax.broadcasted_iota(jnp.int32, (C, H), 0)
    hi = lax.broadcasted_iota(jnp.int32, (C, H), 1)
    bmat = ((ci // DH == hi).astype(jnp.float32)
            * jnp.float32(1.0 / math.sqrt(DH)))
    s0 = jnp.dot(prod, bmat, preferred_element_type=jnp.float32)
    s3 = s0.reshape(nb, K, H)
    mask = mask_ref[...]
    s3 = jnp.where(mask[:, :, None] != 0.0, jnp.float32(-86.0), s3)
    e = jnp.exp(s3)
    attn = e / jnp.sum(e, axis=1, keepdims=True)
    hj = lax.broadcasted_iota(jnp.int32, (H, C), 0)
    cj = lax.broadcasted_iota(jnp.int32, (H, C), 1)
    bmat_t = (cj // DH == hj).astype(jnp.float32)
    ae = jnp.dot(attn.reshape(nb * K, H), bmat_t,
                 preferred_element_type=jnp.float32)
    attend = jnp.sum((ae * vg).reshape(nb, K, C), axis=1)
    x1 = (vf_ref[...]
          + jnp.dot(attend, wo_ref[...], preferred_element_type=jnp.float32)
          + bo_ref[...])
    x1_ref[...] = x1

    @pl.when(pl.program_id(0) == 0)
    def _():
        st_ref[...] = jnp.zeros_like(st_ref)

    st_ref[0:1, :] += jnp.sum(x1, axis=0, keepdims=True)
    st_ref[1:2, :] += jnp.sum(x1 * x1, axis=0, keepdims=True)


def _stage_c(q, kvg, maskf, vf, wo, bo, v_start, v_count):
    nsteps = v_count // _NB_C
    b0 = v_start // _NB_C

    def full(shape):
        return pl.BlockSpec(shape, lambda i: tuple(0 for _ in shape))

    return pl.pallas_call(
        _attn_body,
        grid=(nsteps,),
        in_specs=[
            pl.BlockSpec((_NB_C, C), lambda i: (b0 + i, 0)),
            pl.BlockSpec((_NB_C * K, C), lambda i: (i, 0)),
            pl.BlockSpec((_NB_C, K), lambda i: (b0 + i, 0)),
            pl.BlockSpec((_NB_C, C), lambda i: (b0 + i, 0)),
            full((C, C)), full((1, C)),
        ],
        out_specs=[
            pl.BlockSpec((_NB_C, C), lambda i: (i, 0)),
            full((8, C)),
        ],
        out_shape=[
            jax.ShapeDtypeStruct((v_count, C), jnp.float32),
            jax.ShapeDtypeStruct((8, C), jnp.float32),
        ],
    )(q, kvg, maskf, vf, wo, bo)


_NB_D = 1000


def _ffn_body(x1_ref, s1_ref, g1_ref, be1_ref, w1_ref, b1_ref, w2_ref, b2_ref,
              x2_ref, st_ref):
    invn = jnp.float32(1.0 / N)
    mean = s1_ref[0:1, :] * invn
    var = s1_ref[1:2, :] * invn - mean * mean
    inv = lax.rsqrt(var + 1e-5)
    x1n = (x1_ref[...] - mean) * inv * g1_ref[...] + be1_ref[...]
    hmid = jnp.maximum(
        jnp.dot(x1n, w1_ref[...], preferred_element_type=jnp.float32)
        + b1_ref[...], 0.0)
    ff = (jnp.dot(hmid, w2_ref[...], preferred_element_type=jnp.float32)
          + b2_ref[...])
    x2 = x1n + ff
    x2_ref[...] = x2

    @pl.when(pl.program_id(0) == 0)
    def _():
        st_ref[...] = jnp.zeros_like(st_ref)

    st_ref[0:1, :] += jnp.sum(x2, axis=0, keepdims=True)
    st_ref[1:2, :] += jnp.sum(x2 * x2, axis=0, keepdims=True)


def _stage_d(x1, s1, g1, be1, w1, b1, w2, b2):
    nsteps = N // _NB_D

    def full(shape):
        return pl.BlockSpec(shape, lambda i: tuple(0 for _ in shape))

    return pl.pallas_call(
        _ffn_body,
        grid=(nsteps,),
        in_specs=[
            pl.BlockSpec((_NB_D, C), lambda i: (i, 0)),
            full((8, C)), full((1, C)), full((1, C)),
            full((C, FF)), full((1, FF)),
            full((FF, C)), full((1, C)),
        ],
        out_specs=[
            pl.BlockSpec((_NB_D, C), lambda i: (i, 0)),
            full((8, C)),
        ],
        out_shape=[
            jax.ShapeDtypeStruct((N, C), jnp.float32),
            jax.ShapeDtypeStruct((8, C), jnp.float32),
        ],
    )(x1, s1, g1, be1, w1, b1, w2, b2)


def _outproj_body(x2_ref, s2_ref, g2_ref, be2_ref, wout_ref, bout_ref,
                  y_ref, st_ref):
    invn = jnp.float32(1.0 / N)
    mean = s2_ref[0:1, :] * invn
    var = s2_ref[1:2, :] * invn - mean * mean
    inv = lax.rsqrt(var + 1e-5)
    x2n = (x2_ref[...] - mean) * inv * g2_ref[...] + be2_ref[...]
    y = (jnp.dot(x2n, wout_ref[...], preferred_element_type=jnp.float32)
         + bout_ref[...])
    y_ref[...] = y

    @pl.when(pl.program_id(0) == 0)
    def _():
        st_ref[...] = jnp.zeros_like(st_ref)

    st_ref[0:1, :] += jnp.sum(y, axis=0, keepdims=True)
    st_ref[1:2, :] += jnp.sum(y * y, axis=0, keepdims=True)


def _stage_e(x2, s2, g2, be2, wout, bout):
    nsteps = N // _NB_D

    def full(shape):
        return pl.BlockSpec(shape, lambda i: tuple(0 for _ in shape))

    return pl.pallas_call(
        _outproj_body,
        grid=(nsteps,),
        in_specs=[
            pl.BlockSpec((_NB_D, C), lambda i: (i, 0)),
            full((8, C)), full((1, C)), full((1, C)),
            full((C, OUT)), full((1, OUT)),
        ],
        out_specs=[
            pl.BlockSpec((_NB_D, OUT), lambda i: (i, 0)),
            full((8, OUT)),
        ],
        out_shape=[
            jax.ShapeDtypeStruct((N, OUT), jnp.float32),
            jax.ShapeDtypeStruct((8, OUT), jnp.float32),
        ],
    )(x2, s2, g2, be2, wout, bout)


def _final_body(y_ref, s3_ref, gout_ref, beout_ref, out_ref):
    invn = jnp.float32(1.0 / N)
    mean = s3_ref[0:1, :] * invn
    var = s3_ref[1:2, :] * invn - mean * mean
    inv = lax.rsqrt(var + 1e-5)
    yn = (y_ref[...] - mean) * inv * gout_ref[...] + beout_ref[...]
    out_ref[...] = jnp.maximum(yn, 0.0)


def _stage_f(y, s3, gout, beout):
    nsteps = N // _NB_D

    def full(shape):
        return pl.BlockSpec(shape, lambda i: tuple(0 for _ in shape))

    return pl.pallas_call(
        _final_body,
        grid=(nsteps,),
        in_specs=[
            pl.BlockSpec((_NB_D, OUT), lambda i: (i, 0)),
            full((8, OUT)), full((1, OUT)), full((1, OUT)),
        ],
        out_specs=pl.BlockSpec((_NB_D, OUT), lambda i: (i, 0)),
        out_shape=jax.ShapeDtypeStruct((N, OUT), jnp.float32),
    )(y, s3, gout, beout)


def kernel(voxel_features, voxel_coords, key_indices, key_mask, Wq, bq, Wk, bk,
           Wv, bv, Wo, bo, W1, b1, W2, b2, Wqp, bqp, Wkp, bkp, Wout, bout,
           g1, be1, g2, be2, gout, beout):
    cp = jnp.pad(voxel_coords, ((0, 0), (0, 5)))
    wqp = jnp.pad(Wqp, ((0, 5), (0, 0)))
    wkp = jnp.pad(Wkp, ((0, 5), (0, 0)))
    r = lambda b: b.reshape(1, -1)

    q, kv = _stage_a(voxel_features, cp, wqp, r(bqp), wkp, r(bkp),
                     Wq, r(bq), Wk, r(bk), Wv, r(bv))
    idx = key_indices.reshape(-1).astype(jnp.int32)
    splits = [(0, N)]
    kvgs = [_sc_gather(kv, idx[s * K:(s + c) * K]) for s, c in splits]
    maskf = key_mask.astype(jnp.float32)
    parts = [_stage_c(q, kvg, maskf, voxel_features, Wo, r(bo), s, c)
             for kvg, (s, c) in zip(kvgs, splits)]
    x1 = (parts[0][0] if len(parts) == 1
          else jnp.concatenate([p[0] for p in parts], axis=0))
    s1 = parts[0][1]
    for p in parts[1:]:
        s1 = s1 + p[1]
    x2, s2 = _stage_d(x1, s1, g1.reshape(1, -1), be1.reshape(1, -1),
                      W1, r(b1), W2, r(b2))
    y, s3 = _stage_e(x2, s2, g2.reshape(1, -1), be2.reshape(1, -1),
                     Wout, r(bout))
    return _stage_f(y, s3, gout.reshape(1, -1), beout.reshape(1, -1))
